# Initial kernel scaffold; baseline (speedup 1.0000x reference)
import jax
import jax.numpy as jnp
from jax.experimental import pallas as pl


def kernel(boxes, scores):
    raise NotImplementedError("write your pallas kernel here")


# speedup vs baseline: 105.4644x; 105.4644x over previous
import jax
import jax.numpy as jnp
from jax.experimental import pallas as pl
from jax.experimental.pallas import tpu as pltpu

_NMS_THR = 0.3
_SCORE_THR = 0.1
_NEG = -1.0e30
_NP = 4096
_BLK = 128
_NBLK = _NP // _BLK
_CH = 256
_NCH = _NP // _CH
_MAXP = 100


def _seg_kernel(bc_ref, br_ref, sc_ref, sr_ref, out_ref,
                scol_ref, srow_ref, keep_ref, cum_ref):
    f32 = jnp.float32
    bc = bc_ref[...]
    br = br_ref[...]
    s_col = sc_ref[...]
    s_row = sr_ref[...]

    vc = (s_col > _SCORE_THR) & (bc[:, 2:3] > bc[:, 0:1]) & (bc[:, 3:4] > bc[:, 1:2])
    vr = (s_row > _SCORE_THR) & (br[2:3, :] > br[0:1, :]) & (br[3:4, :] > br[1:2, :])
    m_col = jnp.where(vc, s_col, _NEG)
    m_row = jnp.where(vr, s_row, _NEG)

    j_row = jax.lax.broadcasted_iota(jnp.int32, (1, _NP), 1)

    rank_row = jnp.zeros((1, _NP), f32)
    rank_col_chunks = []
    for t in range(_NCH):
        i0 = t * _CH
        mi = m_col[i0:i0 + _CH, :]
        ii = i0 + jax.lax.broadcasted_iota(jnp.int32, (_CH, 1), 0)
        beats_i = (m_row > mi) | ((m_row == mi) & (j_row < ii))
        rank_col_chunks.append(jnp.sum(beats_i.astype(f32), axis=1, keepdims=True))
        i_beats_j = (~beats_i) & (ii != j_row)
        rank_row = rank_row + jnp.sum(i_beats_j.astype(f32), axis=0, keepdims=True)
    rank_col = jnp.concatenate(rank_col_chunks, axis=0)

    idx_col = jax.lax.broadcasted_iota(jnp.int32, (_NP, 1), 0)
    clsp1_col = ((idx_col & 1) + 1).astype(f32)
    clsp1_row = ((j_row & 1) + 1).astype(f32)
    zero_col = jnp.zeros((_NP, 1), f32)
    x_col = jnp.concatenate([bc, m_col, clsp1_col, zero_col, zero_col], axis=1)
    x_row = jnp.concatenate([br, m_row, clsp1_row, jnp.zeros((2, _NP), f32)], axis=0)

    for t in range(_NCH):
        r0 = t * _CH
        rc = (r0 + jax.lax.broadcasted_iota(jnp.int32, (_CH, 1), 0)).astype(f32)
        p = (rank_row == rc).astype(f32)
        scol_ref[r0:r0 + _CH, :] = jnp.dot(
            p, x_col, preferred_element_type=f32,
            precision=jax.lax.Precision.HIGHEST)
        rr = (r0 + jax.lax.broadcasted_iota(jnp.int32, (1, _CH), 1)).astype(f32)
        pt = (rank_col == rr).astype(f32)
        srow_ref[:, r0:r0 + _CH] = jnp.dot(
            x_row, pt, preferred_element_type=f32,
            precision=jax.lax.Precision.HIGHEST)

    keep_ref[...] = (srow_ref[4:5, :] > -1.0e29).astype(f32)

    tri = (jax.lax.broadcasted_iota(jnp.int32, (_BLK, _BLK), 0)
           < jax.lax.broadcasted_iota(jnp.int32, (_BLK, _BLK), 1))

    rx1 = srow_ref[0:1, :]
    ry1 = srow_ref[1:2, :]
    rx2 = srow_ref[2:3, :]
    ry2 = srow_ref[3:4, :]
    area_r = jnp.maximum(rx2 - rx1, 0.) * jnp.maximum(ry2 - ry1, 0.)

    def nms_block(b, carry):
        base = b * _BLK
        cx1 = scol_ref[pl.ds(base, _BLK), 0:1]
        cy1 = scol_ref[pl.ds(base, _BLK), 1:2]
        cx2 = scol_ref[pl.ds(base, _BLK), 2:3]
        cy2 = scol_ref[pl.ds(base, _BLK), 3:4]
        area_c = jnp.maximum(cx2 - cx1, 0.) * jnp.maximum(cy2 - cy1, 0.)
        iw = jnp.maximum(jnp.minimum(cx2, rx2) - jnp.maximum(cx1, rx1), 0.)
        ih = jnp.maximum(jnp.minimum(cy2, ry2) - jnp.maximum(cy1, ry1), 0.)
        inter = iw * ih
        union = area_c + area_r - inter
        sup_all = (inter > _NMS_THR * jnp.maximum(union, 1e-9)).astype(f32)

        rbx1 = srow_ref[0:1, pl.ds(base, _BLK)]
        rby1 = srow_ref[1:2, pl.ds(base, _BLK)]
        rbx2 = srow_ref[2:3, pl.ds(base, _BLK)]
        rby2 = srow_ref[3:4, pl.ds(base, _BLK)]
        iwb = jnp.maximum(jnp.minimum(cx2, rbx2) - jnp.maximum(cx1, rbx1), 0.)
        ihb = jnp.maximum(jnp.minimum(cy2, rby2) - jnp.maximum(cy1, rby1), 0.)
        interb = iwb * ihb
        area_rb = jnp.maximum(rbx2 - rbx1, 0.) * jnp.maximum(rby2 - rby1, 0.)
        unionb = area_c + area_rb - interb
        s_blk = ((interb > _NMS_THR * jnp.maximum(unionb, 1e-9)) & tri).astype(f32)

        kb = keep_ref[0:1, pl.ds(base, _BLK)]

        def fp_cond(c):
            return c[1]

        def fp_body(c):
            k, _ = c
            sup = jnp.dot(k, s_blk, preferred_element_type=f32) > 0.
            kn = jnp.where(sup, 0., kb)
            return kn, jnp.any(kn != k)

        kfin, _ = jax.lax.while_loop(fp_cond, fp_body, (kb, True))

        keep_ref[0:1, pl.ds(base, _BLK)] = kfin
        sup_later = jnp.dot(kfin, sup_all, preferred_element_type=f32) > 0.
        later = j_row >= base + _BLK
        k_all = keep_ref[0:1, :]
        keep_ref[0:1, :] = jnp.where(sup_later & later, 0., k_all)
        return carry

    jax.lax.fori_loop(0, _NBLK, nms_block, 0)

    triu = (jax.lax.broadcasted_iota(jnp.int32, (_BLK, _BLK), 0)
            <= jax.lax.broadcasted_iota(jnp.int32, (_BLK, _BLK), 1)).astype(f32)

    def cum_body(c, off):
        kch = keep_ref[0:1, pl.ds(c * _BLK, _BLK)]
        cum_ref[0:1, pl.ds(c * _BLK, _BLK)] = (
            jnp.dot(kch, triu, preferred_element_type=f32) + off)
        return off + jnp.sum(kch)

    jax.lax.fori_loop(0, _NBLK, cum_body, jnp.float32(0.))

    cum = cum_ref[...]
    keepv = keep_ref[...] > 0.
    r_iota = jax.lax.broadcasted_iota(jnp.int32, (_BLK, 1), 0).astype(f32)
    q = (keepv & ((cum - 1.) == r_iota)).astype(f32)
    out_ref[...] = jnp.dot(q, scol_ref[...], preferred_element_type=f32,
                           precision=jax.lax.Precision.HIGHEST)


def _run(bc, br, sc, sr, interpret=False):
    B = bc.shape[0]
    return pl.pallas_call(
        _seg_kernel,
        grid=(B,),
        in_specs=[
            pl.BlockSpec((None, _NP, 4), lambda b: (b, 0, 0)),
            pl.BlockSpec((None, 4, _NP), lambda b: (b, 0, 0)),
            pl.BlockSpec((None, _NP, 1), lambda b: (b, 0, 0)),
            pl.BlockSpec((None, 1, _NP), lambda b: (b, 0, 0)),
        ],
        out_specs=pl.BlockSpec((None, _BLK, 8), lambda b: (b, 0, 0)),
        out_shape=jax.ShapeDtypeStruct((B, _BLK, 8), jnp.float32),
        scratch_shapes=[
            pltpu.VMEM((_NP, 8), jnp.float32),
            pltpu.VMEM((8, _NP), jnp.float32),
            pltpu.VMEM((1, _NP), jnp.float32),
            pltpu.VMEM((1, _NP), jnp.float32),
        ],
        interpret=interpret,
    )(bc, br, sc, sr)


def kernel(boxes, scores):
    B, N, C = boxes.shape[0], boxes.shape[1], boxes.shape[2]
    M = N * C
    bf = boxes.reshape(B, M, 4)
    sf = scores[:, :, 1:].reshape(B, M)
    pad = _NP - M
    bf = jnp.pad(bf, ((0, 0), (0, pad), (0, 0)))
    sf = jnp.pad(sf, ((0, 0), (0, pad)), constant_values=-1.0)
    out = _run(bf, jnp.swapaxes(bf, 1, 2), sf[:, :, None], sf[:, None, :])
    bbx = out[:, :_MAXP, :4]
    obj = out[:, :_MAXP, 4]
    cls = (out[:, :_MAXP, 5] - 1.0).astype(jnp.int32)
    return bbx, obj, cls
